# baseline (speedup 1.0000x reference)
import functools

import jax
import jax.numpy as jnp
from jax import lax
from jax.experimental import pallas as pl
from jax.experimental.pallas import tpu as pltpu
from jax.experimental.pallas import tpu_sc as plsc

LANES = 16
NC, NS = 2, 16
NW = NC * NS
CHUNK = 100
NBUF = 4
UNROLL = 8


@functools.partial(jax.jit, static_argnames=("n_rows", "l_len", "d_model"))
def _sc_embed(ids2d, table, pos, n_rows, l_len, d_model):
    group = l_len
    rows_per_w = n_rows // NW
    groups_per_w = rows_per_w // group
    chunks_per_w = rows_per_w // CHUNK
    chunks_per_g = group // CHUNK
    mesh = plsc.VectorSubcoreMesh(core_axis_name="c", subcore_axis_name="s")

    @functools.partial(
        pl.kernel,
        out_type=jax.ShapeDtypeStruct((n_rows, d_model), jnp.float32),
        mesh=mesh,
        scratch_types=(
            [pltpu.VMEM((chunks_per_w, CHUNK), jnp.int32),
             pltpu.VMEM((group, d_model), jnp.float32)]
            + [pltpu.VMEM((group, d_model), jnp.float32) for _ in range(NBUF)]
            + [pltpu.SemaphoreType.DMA for _ in range(2 * NBUF)]
        ),
        compiler_params=pltpu.CompilerParams(use_tc_tiling_on_sc=False),
    )
    def k(ids_hbm, table_hbm, pos_hbm, out_hbm, idx_v, pos_v, *rest):
        bufs = rest[:NBUF]
        gsems = rest[NBUF:2 * NBUF]
        osems = rest[2 * NBUF:]
        wid = lax.axis_index("s") * NC + lax.axis_index("c")
        base_row = wid * rows_per_w

        pltpu.sync_copy(ids_hbm.at[pl.ds(wid * chunks_per_w, chunks_per_w)],
                        idx_v)
        pltpu.sync_copy(pos_hbm, pos_v)

        def start_gather(g, b):
            for j in range(chunks_per_g):
                c = g * chunks_per_g + j
                pltpu.async_copy(table_hbm.at[idx_v.at[c]],
                                 bufs[b].at[pl.ds(j * CHUNK, CHUNK)],
                                 gsems[b])

        def wait_gather(b):
            pltpu.make_async_copy(pos_hbm, bufs[b], gsems[b]).wait()

        def wait_write(b):
            pltpu.make_async_copy(bufs[b],
                                  out_hbm.at[pl.ds(0, group)],
                                  osems[b]).wait()

        start_gather(0, 0)
        start_gather(1, 1)

        def outer(og, carry):
            for b in range(NBUF):
                g = og * NBUF + b
                wait_gather(b)

                def row_body(i, _):
                    for u in range(UNROLL):
                        r = i * UNROLL + u
                        for c in range(d_model // LANES):
                            plsc.addupdate(
                                bufs[b].at[r, pl.ds(c * LANES, LANES)],
                                pos_v[r, pl.ds(c * LANES, LANES)])
                    return 0

                lax.fori_loop(0, group // UNROLL, row_body, 0,
                              unroll=False)
                pltpu.async_copy(
                    bufs[b],
                    out_hbm.at[pl.ds(base_row + g * group, group)],
                    osems[b])

                h = g + 2
                hb = (b + 2) % NBUF

                @pl.when(h < groups_per_w)
                def _():
                    @pl.when(h >= NBUF)
                    def _():
                        wait_write(hb)
                    start_gather(h, hb)
            return carry

        lax.fori_loop(0, groups_per_w // NBUF, outer, 0, unroll=False)

    return k(ids2d, table, pos)


def kernel(input_ids, token_embed, positional):
    b, l = input_ids.shape
    d = token_embed.shape[1]
    n = b * l
    ids2d = input_ids.reshape(n // CHUNK, CHUNK).astype(jnp.int32)
    pos = positional[:l]
    out = _sc_embed(ids2d, token_embed, pos, n, l, d)
    return out.reshape(b, l, d)


# speedup vs baseline: 1.0009x; 1.0009x over previous
import functools

import jax
import jax.numpy as jnp
from jax import lax
from jax.experimental import pallas as pl
from jax.experimental.pallas import tpu as pltpu
from jax.experimental.pallas import tpu_sc as plsc

LANES = 16
NC, NS = 2, 16
NW = NC * NS
CHUNK0 = 128
NBUF = 4
UNROLL = 8


@functools.partial(jax.jit, static_argnames=("n_batch", "l_len", "d_model"))
def _sc_embed(ids, table, pos, n_batch, l_len, d_model):
    group = l_len
    batches_per_w = n_batch // NW
    chunk1 = group - CHUNK0
    mesh = plsc.VectorSubcoreMesh(core_axis_name="c", subcore_axis_name="s")

    @functools.partial(
        pl.kernel,
        out_type=jax.ShapeDtypeStruct((n_batch * group, d_model),
                                      jnp.float32),
        mesh=mesh,
        scratch_types=(
            [pltpu.VMEM((batches_per_w, group), jnp.int32),
             pltpu.VMEM((group, d_model), jnp.float32)]
            + [pltpu.VMEM((group, d_model), jnp.float32) for _ in range(NBUF)]
            + [pltpu.SemaphoreType.DMA for _ in range(2 * NBUF)]
        ),
        compiler_params=pltpu.CompilerParams(use_tc_tiling_on_sc=False),
    )
    def k(ids_hbm, table_hbm, pos_hbm, out_hbm, idx_v, pos_v, *rest):
        bufs = rest[:NBUF]
        gsems = rest[NBUF:2 * NBUF]
        osems = rest[2 * NBUF:]
        wid = lax.axis_index("s") * NC + lax.axis_index("c")
        base_row = wid * batches_per_w * group

        pltpu.sync_copy(ids_hbm.at[pl.ds(wid * batches_per_w, batches_per_w)],
                        idx_v)
        pltpu.sync_copy(pos_hbm, pos_v)

        def start_gather(g, b):
            pltpu.async_copy(table_hbm.at[idx_v.at[g, pl.ds(0, CHUNK0)]],
                             bufs[b].at[pl.ds(0, CHUNK0)],
                             gsems[b])
            pltpu.async_copy(table_hbm.at[idx_v.at[g, pl.ds(CHUNK0, chunk1)]],
                             bufs[b].at[pl.ds(CHUNK0, chunk1)],
                             gsems[b])

        def wait_gather(b):
            pltpu.make_async_copy(pos_hbm, bufs[b], gsems[b]).wait()

        def wait_write(b):
            pltpu.make_async_copy(bufs[b],
                                  out_hbm.at[pl.ds(0, group)],
                                  osems[b]).wait()

        start_gather(0, 0)
        start_gather(1, 1)

        def outer(og, carry):
            for b in range(NBUF):
                g = og * NBUF + b
                wait_gather(b)

                def row_body(i, _):
                    for u in range(UNROLL):
                        r = i * UNROLL + u
                        for c in range(d_model // LANES):
                            plsc.addupdate(
                                bufs[b].at[r, pl.ds(c * LANES, LANES)],
                                pos_v[r, pl.ds(c * LANES, LANES)])
                    return 0

                lax.fori_loop(0, group // UNROLL, row_body, 0,
                              unroll=False)
                pltpu.async_copy(
                    bufs[b],
                    out_hbm.at[pl.ds(base_row + g * group, group)],
                    osems[b])

                h = g + 2
                hb = (b + 2) % NBUF

                @pl.when(h < batches_per_w)
                def _():
                    @pl.when(h >= NBUF)
                    def _():
                        wait_write(hb)
                    start_gather(h, hb)
            return carry

        lax.fori_loop(0, batches_per_w // NBUF, outer, 0, unroll=False)

    return k(ids, table, pos)


def kernel(input_ids, token_embed, positional):
    b, l = input_ids.shape
    d = token_embed.shape[1]
    ids = input_ids.astype(jnp.int32)
    pos = positional[:l]
    out = _sc_embed(ids, token_embed, pos, b, l, d)
    return out.reshape(b, l, d)
